# baseline (speedup 1.0000x reference)
import functools

import jax
import jax.numpy as jnp
from jax import lax
from jax.experimental import pallas as pl
from jax.experimental.pallas import tpu as pltpu
from jax.experimental.pallas import tpu_sc as plsc

N_VOCAB = 1000
HIDDEN = 128
BATCH = 1024
SEQ = 50
TOK = BATCH * SEQ

NC = 2
NS = 16
NW = NC * NS
B_PER_W = TOK // NW
CHUNK = 400
NCHUNK = B_PER_W // CHUNK


def _sc_gather(table, idx):
    mesh = plsc.VectorSubcoreMesh(core_axis_name="c", subcore_axis_name="s")

    @functools.partial(
        pl.kernel,
        out_type=jax.ShapeDtypeStruct((TOK, HIDDEN), jnp.float32),
        mesh=mesh,
        scratch_types=[
            pltpu.VMEM((CHUNK,), jnp.int32),
            pltpu.VMEM((CHUNK, HIDDEN), jnp.float32),
            pltpu.SemaphoreType.DMA,
        ],
    )
    def k(table_hbm, idx_hbm, out_hbm, idx_v, rows_v, sem):
        wid = lax.axis_index("s") * NC + lax.axis_index("c")
        base = wid * B_PER_W
        for c in range(NCHUNK):
            off = base + c * CHUNK
            pltpu.sync_copy(idx_hbm.at[pl.ds(off, CHUNK)], idx_v)
            pltpu.async_copy(table_hbm.at[idx_v], rows_v, sem).wait()
            pltpu.sync_copy(rows_v, out_hbm.at[pl.ds(off, CHUNK)])

    return k(table, idx)


TB = 512


def _tc_mlp(e, W1, b1, W2, b2):
    def body(e_ref, w1_ref, b1_ref, w2_ref, b2_ref, out_ref):
        h = jnp.dot(e_ref[...], w1_ref[...], preferred_element_type=jnp.float32)
        h = jnp.maximum(h + b1_ref[...], 0.0)
        out_ref[...] = (
            jnp.dot(h, w2_ref[...], preferred_element_type=jnp.float32)
            + b2_ref[...]
        )

    return pl.pallas_call(
        body,
        grid=(TOK // TB,),
        in_specs=[
            pl.BlockSpec((TB, HIDDEN), lambda i: (i, 0)),
            pl.BlockSpec((HIDDEN, HIDDEN), lambda i: (0, 0)),
            pl.BlockSpec((1, HIDDEN), lambda i: (0, 0)),
            pl.BlockSpec((HIDDEN, N_VOCAB), lambda i: (0, 0)),
            pl.BlockSpec((1, N_VOCAB), lambda i: (0, 0)),
        ],
        out_specs=pl.BlockSpec((TB, N_VOCAB), lambda i: (i, 0)),
        out_shape=jax.ShapeDtypeStruct((TOK, N_VOCAB), jnp.float32),
        compiler_params=pltpu.CompilerParams(
            dimension_semantics=("parallel",),
        ),
    )(e, W1, b1, W2, b2)


def kernel(x, embed, W1, b1, W2, b2):
    idx = x.reshape(TOK).astype(jnp.int32)
    e = _sc_gather(embed, idx)
    logits = _tc_mlp(e, W1, b1.reshape(1, HIDDEN), W2, b2.reshape(1, N_VOCAB))
    return logits.reshape(BATCH, SEQ, N_VOCAB)


# speedup vs baseline: 1.1853x; 1.1853x over previous
import functools

import jax
import jax.numpy as jnp
from jax import lax
from jax.experimental import pallas as pl
from jax.experimental.pallas import tpu as pltpu
from jax.experimental.pallas import tpu_sc as plsc

N_VOCAB = 1000
HIDDEN = 128
BATCH = 1024
SEQ = 50
TOK = BATCH * SEQ

NC = 2
NS = 16
NW = NC * NS
B_PER_W = TOK // NW
CHUNK = 400
NCHUNK = B_PER_W // CHUNK


def _sc_gather(table, idx):
    mesh = plsc.VectorSubcoreMesh(core_axis_name="c", subcore_axis_name="s")

    @functools.partial(
        pl.kernel,
        out_type=jax.ShapeDtypeStruct((TOK, HIDDEN), jnp.float32),
        mesh=mesh,
        scratch_types=[
            pltpu.VMEM((CHUNK,), jnp.int32),
            pltpu.VMEM((CHUNK, HIDDEN), jnp.float32),
            pltpu.SemaphoreType.DMA,
        ],
    )
    def k(table_hbm, idx_hbm, out_hbm, idx_v, rows_v, sem):
        wid = lax.axis_index("s") * NC + lax.axis_index("c")
        base = wid * B_PER_W
        for c in range(NCHUNK):
            off = base + c * CHUNK
            pltpu.sync_copy(idx_hbm.at[pl.ds(off, CHUNK)], idx_v)
            pltpu.async_copy(table_hbm.at[idx_v], rows_v, sem).wait()
            pltpu.sync_copy(rows_v, out_hbm.at[pl.ds(off, CHUNK)])

    return k(table, idx)


BB = 8
TB = BB * SEQ


def _tc_mlp(e, W1, b1, W2, b2):
    def body(e_ref, w1_ref, b1_ref, w2_ref, b2_ref, out_ref):
        eb = e_ref[...].astype(jnp.bfloat16)
        h = jnp.dot(eb, w1_ref[...], preferred_element_type=jnp.float32)
        h = jnp.maximum(h + b1_ref[...], 0.0).astype(jnp.bfloat16)
        for b in range(BB):
            hb = h[b * SEQ:(b + 1) * SEQ, :]
            out_ref[b] = (
                jnp.dot(hb, w2_ref[...], preferred_element_type=jnp.float32)
                + b2_ref[...]
            )

    return pl.pallas_call(
        body,
        grid=(BATCH // BB,),
        in_specs=[
            pl.BlockSpec((TB, HIDDEN), lambda i: (i, 0)),
            pl.BlockSpec((HIDDEN, HIDDEN), lambda i: (0, 0)),
            pl.BlockSpec((1, HIDDEN), lambda i: (0, 0)),
            pl.BlockSpec((HIDDEN, N_VOCAB), lambda i: (0, 0)),
            pl.BlockSpec((1, N_VOCAB), lambda i: (0, 0)),
        ],
        out_specs=pl.BlockSpec((BB, SEQ, N_VOCAB), lambda i: (i, 0, 0)),
        out_shape=jax.ShapeDtypeStruct((BATCH, SEQ, N_VOCAB), jnp.float32),
        compiler_params=pltpu.CompilerParams(
            dimension_semantics=("parallel",),
        ),
    )(e, W1, b1, W2, b2)


def kernel(x, embed, W1, b1, W2, b2):
    idx = x.reshape(TOK).astype(jnp.int32)
    e = _sc_gather(embed, idx)
    return _tc_mlp(
        e,
        W1.astype(jnp.bfloat16),
        b1.reshape(1, HIDDEN),
        W2.astype(jnp.bfloat16),
        b2.reshape(1, N_VOCAB),
    )


# speedup vs baseline: 3.6474x; 3.0772x over previous
import functools

import jax
import jax.numpy as jnp
from jax import lax
from jax.experimental import pallas as pl
from jax.experimental.pallas import tpu as pltpu
from jax.experimental.pallas import tpu_sc as plsc

N_VOCAB = 1000
HIDDEN = 128
BATCH = 1024
SEQ = 50
TOK = BATCH * SEQ

NC = 2
NS = 16
NW = NC * NS
B_PER_W = TOK // NW
CHUNK = 400
NCHUNK = B_PER_W // CHUNK


def _sc_gather(table, idx):
    mesh = plsc.VectorSubcoreMesh(core_axis_name="c", subcore_axis_name="s")

    @functools.partial(
        pl.kernel,
        out_type=jax.ShapeDtypeStruct((TOK, HIDDEN), jnp.float32),
        mesh=mesh,
        scratch_types=[
            pltpu.VMEM((CHUNK,), jnp.int32),
            pltpu.VMEM((CHUNK, HIDDEN), jnp.float32),
            pltpu.SemaphoreType.DMA,
        ],
    )
    def k(table_hbm, idx_hbm, out_hbm, idx_v, rows_v, sem):
        wid = lax.axis_index("s") * NC + lax.axis_index("c")
        base = wid * B_PER_W
        for c in range(NCHUNK):
            off = base + c * CHUNK
            pltpu.sync_copy(idx_hbm.at[pl.ds(off, CHUNK)], idx_v)
            pltpu.async_copy(table_hbm.at[idx_v], rows_v, sem).wait()
            pltpu.sync_copy(rows_v, out_hbm.at[pl.ds(off, CHUNK)])

    return k(table, idx)


NB = 256
NI = BATCH // NB
VB = 200
NJ = N_VOCAB // VB


def _tc_mlp(e3, W1, b1, W2T, b2c):
    def body(e_ref, w1_ref, b1_ref, w2t_ref, b2_ref, out_ref, h_ref):
        j = pl.program_id(1)

        @pl.when(j == 0)
        def _():
            w1 = w1_ref[...]
            b1v = b1_ref[...]
            for s in range(SEQ):
                es = e_ref[s].astype(jnp.bfloat16)
                h = jnp.dot(es, w1, preferred_element_type=jnp.float32)
                h_ref[pl.ds(s * NB, NB), :] = jnp.maximum(
                    h + b1v, 0.0
                ).astype(jnp.bfloat16)

        w2t = w2t_ref[...]
        b2v = b2_ref[...]
        for s in range(SEQ):
            hs = h_ref[pl.ds(s * NB, NB), :]
            out_ref[s] = (
                lax.dot_general(
                    w2t, hs, (((1,), (1,)), ((), ())),
                    preferred_element_type=jnp.float32,
                )
                + b2v
            )

    return pl.pallas_call(
        body,
        grid=(NI, NJ),
        in_specs=[
            pl.BlockSpec((SEQ, NB, HIDDEN), lambda i, j: (0, i, 0)),
            pl.BlockSpec((HIDDEN, HIDDEN), lambda i, j: (0, 0)),
            pl.BlockSpec((1, HIDDEN), lambda i, j: (0, 0)),
            pl.BlockSpec((VB, HIDDEN), lambda i, j: (j, 0)),
            pl.BlockSpec((VB, 1), lambda i, j: (j, 0)),
        ],
        out_specs=pl.BlockSpec((SEQ, VB, NB), lambda i, j: (0, j, i)),
        out_shape=jax.ShapeDtypeStruct((SEQ, N_VOCAB, BATCH), jnp.float32),
        scratch_shapes=[pltpu.VMEM((SEQ * NB, HIDDEN), jnp.bfloat16)],
        compiler_params=pltpu.CompilerParams(
            dimension_semantics=("parallel", "arbitrary"),
        ),
    )(e3, W1, b1, W2T, b2c)


def kernel(x, embed, W1, b1, W2, b2):
    idx = x.T.reshape(TOK).astype(jnp.int32)
    e = _sc_gather(embed, idx)
    e3 = e.reshape(SEQ, BATCH, HIDDEN)
    out = _tc_mlp(
        e3,
        W1.astype(jnp.bfloat16),
        b1.reshape(1, HIDDEN),
        W2.T.astype(jnp.bfloat16),
        b2.reshape(N_VOCAB, 1),
    )
    return jnp.transpose(out, (2, 0, 1))


# speedup vs baseline: 3.7715x; 1.0340x over previous
import functools

import jax
import jax.numpy as jnp
from jax import lax
from jax.experimental import pallas as pl
from jax.experimental.pallas import tpu as pltpu
from jax.experimental.pallas import tpu_sc as plsc

N_VOCAB = 1000
HIDDEN = 128
BATCH = 1024
SEQ = 50
TOK = BATCH * SEQ

NC = 2
NS = 16
NW = NC * NS
B_PER_W = TOK // NW
CHUNK = 400
NCHUNK = B_PER_W // CHUNK


def _sc_gather(table, idx):
    mesh = plsc.VectorSubcoreMesh(core_axis_name="c", subcore_axis_name="s")

    @functools.partial(
        pl.kernel,
        out_type=jax.ShapeDtypeStruct((TOK, HIDDEN), jnp.float32),
        mesh=mesh,
        scratch_types=[
            pltpu.VMEM((CHUNK,), jnp.int32),
            pltpu.VMEM((CHUNK, HIDDEN), jnp.float32),
            pltpu.SemaphoreType.DMA,
        ],
    )
    def k(table_hbm, idx_hbm, out_hbm, idx_v, rows_v, sem):
        wid = lax.axis_index("s") * NC + lax.axis_index("c")
        base = wid * B_PER_W
        for c in range(NCHUNK):
            off = base + c * CHUNK
            pltpu.sync_copy(idx_hbm.at[pl.ds(off, CHUNK)], idx_v)
            pltpu.async_copy(table_hbm.at[idx_v], rows_v, sem).wait()
            pltpu.sync_copy(rows_v, out_hbm.at[pl.ds(off, CHUNK)])

    return k(table, idx)


NB = 256
NCH = BATCH // NB
SB = 2


def _tc_mlp(e3, W1, b1, W2T, b2c):
    def body(e_ref, w1_ref, b1_ref, w2t_ref, b2_ref, out_ref):
        w1 = w1_ref[...]
        b1v = b1_ref[...]
        w2t = w2t_ref[...]
        b2v = b2_ref[...]
        for s in range(SB):
            for c in range(NCH):
                es = e_ref[s, pl.ds(c * NB, NB), :].astype(jnp.bfloat16)
                h = jnp.dot(es, w1, preferred_element_type=jnp.float32)
                h = jnp.maximum(h + b1v, 0.0).astype(jnp.bfloat16)
                out_ref[s, :, pl.ds(c * NB, NB)] = (
                    lax.dot_general(
                        w2t, h, (((1,), (1,)), ((), ())),
                        preferred_element_type=jnp.float32,
                    )
                    + b2v
                )

    return pl.pallas_call(
        body,
        grid=(SEQ // SB,),
        in_specs=[
            pl.BlockSpec((SB, BATCH, HIDDEN), lambda s: (s, 0, 0)),
            pl.BlockSpec((HIDDEN, HIDDEN), lambda s: (0, 0)),
            pl.BlockSpec((1, HIDDEN), lambda s: (0, 0)),
            pl.BlockSpec((N_VOCAB, HIDDEN), lambda s: (0, 0)),
            pl.BlockSpec((N_VOCAB, 1), lambda s: (0, 0)),
        ],
        out_specs=pl.BlockSpec((SB, N_VOCAB, BATCH), lambda s: (s, 0, 0)),
        out_shape=jax.ShapeDtypeStruct((SEQ, N_VOCAB, BATCH), jnp.float32),
        compiler_params=pltpu.CompilerParams(
            dimension_semantics=("parallel",),
        ),
    )(e3, W1, b1, W2T, b2c)


def kernel(x, embed, W1, b1, W2, b2):
    idx = x.T.reshape(TOK).astype(jnp.int32)
    e = _sc_gather(embed, idx)
    e3 = e.reshape(SEQ, BATCH, HIDDEN)
    out = _tc_mlp(
        e3,
        W1.astype(jnp.bfloat16),
        b1.reshape(1, HIDDEN),
        W2.T.astype(jnp.bfloat16),
        b2.reshape(N_VOCAB, 1),
    )
    return jnp.transpose(out, (2, 0, 1))


# speedup vs baseline: 3.8258x; 1.0144x over previous
import functools

import jax
import jax.numpy as jnp
from jax import lax
from jax.experimental import pallas as pl
from jax.experimental.pallas import tpu as pltpu
from jax.experimental.pallas import tpu_sc as plsc

N_VOCAB = 1000
HIDDEN = 128
BATCH = 1024
SEQ = 50
TOK = BATCH * SEQ

NC = 2
NS = 16
NW = NC * NS
B_PER_W = TOK // NW
CHUNK = 400
NCHUNK = B_PER_W // CHUNK


def _sc_gather(table, idx):
    mesh = plsc.VectorSubcoreMesh(core_axis_name="c", subcore_axis_name="s")

    @functools.partial(
        pl.kernel,
        out_type=jax.ShapeDtypeStruct((TOK, HIDDEN), jnp.float32),
        mesh=mesh,
        scratch_types=[
            pltpu.VMEM((CHUNK,), jnp.int32),
            pltpu.VMEM((CHUNK,), jnp.int32),
            pltpu.VMEM((CHUNK, HIDDEN), jnp.float32),
            pltpu.VMEM((CHUNK, HIDDEN), jnp.float32),
            pltpu.SemaphoreType.DMA,
            pltpu.SemaphoreType.DMA,
            pltpu.SemaphoreType.DMA,
            pltpu.SemaphoreType.DMA,
        ],
    )
    def k(table_hbm, idx_hbm, out_hbm,
          idx0, idx1, rows0, rows1, g0, g1, w0, w1):
        wid = lax.axis_index("s") * NC + lax.axis_index("c")
        base = wid * B_PER_W
        idx_v = [idx0, idx1]
        rows_v = [rows0, rows1]
        gsem = [g0, g1]
        wsem = [w0, w1]
        pltpu.sync_copy(idx_hbm.at[pl.ds(base, CHUNK)], idx0)
        gathers = [pltpu.async_copy(table_hbm.at[idx0], rows0, g0), None]
        writes = [None, None]
        for c in range(NCHUNK):
            cur, nxt = c % 2, (c + 1) % 2
            if c + 1 < NCHUNK:
                if writes[nxt] is not None:
                    writes[nxt].wait()
                pltpu.sync_copy(
                    idx_hbm.at[pl.ds(base + (c + 1) * CHUNK, CHUNK)],
                    idx_v[nxt],
                )
                gathers[nxt] = pltpu.async_copy(
                    table_hbm.at[idx_v[nxt]], rows_v[nxt], gsem[nxt]
                )
            gathers[cur].wait()
            writes[cur] = pltpu.async_copy(
                rows_v[cur], out_hbm.at[pl.ds(base + c * CHUNK, CHUNK)],
                wsem[cur],
            )
        writes[0].wait()
        writes[1].wait()

    return k(table, idx)


NB = 256
NCH = BATCH // NB
SB = 2


def _tc_mlp(e3, W1, b1, W2T, b2c):
    def body(e_ref, w1_ref, b1_ref, w2t_ref, b2_ref, out_ref):
        w1 = w1_ref[...]
        b1v = b1_ref[...]
        w2t = w2t_ref[...]
        b2v = b2_ref[...]
        for s in range(SB):
            for c in range(NCH):
                es = e_ref[s, pl.ds(c * NB, NB), :].astype(jnp.bfloat16)
                h = jnp.dot(es, w1, preferred_element_type=jnp.float32)
                h = jnp.maximum(h + b1v, 0.0).astype(jnp.bfloat16)
                out_ref[s, :, pl.ds(c * NB, NB)] = (
                    lax.dot_general(
                        w2t, h, (((1,), (1,)), ((), ())),
                        preferred_element_type=jnp.float32,
                    )
                    + b2v
                )

    return pl.pallas_call(
        body,
        grid=(SEQ // SB,),
        in_specs=[
            pl.BlockSpec((SB, BATCH, HIDDEN), lambda s: (s, 0, 0)),
            pl.BlockSpec((HIDDEN, HIDDEN), lambda s: (0, 0)),
            pl.BlockSpec((1, HIDDEN), lambda s: (0, 0)),
            pl.BlockSpec((N_VOCAB, HIDDEN), lambda s: (0, 0)),
            pl.BlockSpec((N_VOCAB, 1), lambda s: (0, 0)),
        ],
        out_specs=pl.BlockSpec((SB, N_VOCAB, BATCH), lambda s: (s, 0, 0)),
        out_shape=jax.ShapeDtypeStruct((SEQ, N_VOCAB, BATCH), jnp.float32),
        compiler_params=pltpu.CompilerParams(
            dimension_semantics=("parallel",),
        ),
    )(e3, W1, b1, W2T, b2c)


def kernel(x, embed, W1, b1, W2, b2):
    idx = x.T.reshape(TOK).astype(jnp.int32)
    e = _sc_gather(embed, idx)
    e3 = e.reshape(SEQ, BATCH, HIDDEN)
    out = _tc_mlp(
        e3,
        W1.astype(jnp.bfloat16),
        b1.reshape(1, HIDDEN),
        W2.T.astype(jnp.bfloat16),
        b2.reshape(N_VOCAB, 1),
    )
    return jnp.transpose(out, (2, 0, 1))
